# baseline (speedup 1.0000x reference)
import functools

import jax
import jax.numpy as jnp
from jax import lax
from jax.experimental import pallas as pl
from jax.experimental.pallas import tpu as pltpu
from jax.experimental.pallas import tpu_sc as plsc

DIM = 1024
HIDDEN = 2752
E = 8
T = 4096

BLK = 512
TILE = 512
R_TILES = T * 2 // TILE + E
R_ROWS = R_TILES * TILE
S_TILES = T // TILE

NC, NS, L = 2, 16, 16
NW = NC * NS


def _gate_body(x_ref, gw_ref, xb_ref, e0_ref, e1_ref, r0_ref, r1_ref,
               w0_ref, w1_ref, cnt_ref, carry):
    i = pl.program_id(0)

    @pl.when(i == 0)
    def _():
        carry[...] = jnp.zeros_like(carry)

    xb = x_ref[...]
    xb_ref[...] = xb.astype(jnp.bfloat16)
    logits = lax.dot_general(xb, gw_ref[...], (((1,), (1,)), ((), ())),
                             preferred_element_type=jnp.float32)
    m = jnp.max(logits, axis=1, keepdims=True)
    ex = jnp.exp(logits - m)
    scores = ex / jnp.sum(ex, axis=1, keepdims=True)
    lane = lax.broadcasted_iota(jnp.int32, scores.shape, 1)
    rank = jnp.zeros(scores.shape, jnp.int32)
    for ep in range(E):
        sc = lax.slice_in_dim(scores, ep, ep + 1, axis=1)
        beats = (sc > scores) | ((sc == scores) & (ep < lane))
        rank = rank + beats.astype(jnp.int32)
    is0 = rank == 0
    is1 = rank == 1
    e0 = jnp.sum(jnp.where(is0, lane, 0), axis=1, keepdims=True)
    e1 = jnp.sum(jnp.where(is1, lane, 0), axis=1, keepdims=True)
    w0 = jnp.sum(jnp.where(is0, scores, 0.0), axis=1, keepdims=True)
    w1 = jnp.sum(jnp.where(is1, scores, 0.0), axis=1, keepdims=True)

    oh = (is0 | is1).astype(jnp.float32)
    tri = (lax.broadcasted_iota(jnp.int32, (BLK, BLK), 0)
           > lax.broadcasted_iota(jnp.int32, (BLK, BLK), 1)).astype(jnp.float32)
    excl = lax.dot_general(tri, oh, (((1,), (0,)), ((), ())),
                           preferred_element_type=jnp.float32)
    cbase = carry[0:1, 0:E]
    r_both = excl + cbase
    r0 = jnp.sum(jnp.where(is0, r_both, 0.0), axis=1, keepdims=True)
    r1 = jnp.sum(jnp.where(is1, r_both, 0.0), axis=1, keepdims=True)

    new_carry = cbase + jnp.sum(oh, axis=0, keepdims=True)
    carry[...] = jnp.pad(new_carry, ((0, 0), (0, 128 - E)))
    cnt_ref[...] = carry[...]

    e0_ref[...] = e0.astype(jnp.int32).reshape(1, BLK // 128, 128)
    e1_ref[...] = e1.astype(jnp.int32).reshape(1, BLK // 128, 128)
    r0_ref[...] = r0.astype(jnp.int32).reshape(1, BLK // 128, 128)
    r1_ref[...] = r1.astype(jnp.int32).reshape(1, BLK // 128, 128)
    w0_ref[...] = jnp.broadcast_to(w0, (BLK, 16))
    w1_ref[...] = jnp.broadcast_to(w1, (BLK, 16))


def _gate(xf, gate_w):
    nblk = T // BLK
    sub = BLK // 128
    return pl.pallas_call(
        _gate_body,
        grid=(nblk,),
        in_specs=[
            pl.BlockSpec((BLK, DIM), lambda i: (i, 0)),
            pl.BlockSpec((E, DIM), lambda i: (0, 0)),
        ],
        out_specs=[
            pl.BlockSpec((BLK, DIM), lambda i: (i, 0)),
            pl.BlockSpec((1, sub, 128), lambda i: (i, 0, 0)),
            pl.BlockSpec((1, sub, 128), lambda i: (i, 0, 0)),
            pl.BlockSpec((1, sub, 128), lambda i: (i, 0, 0)),
            pl.BlockSpec((1, sub, 128), lambda i: (i, 0, 0)),
            pl.BlockSpec((BLK, 16), lambda i: (i, 0)),
            pl.BlockSpec((BLK, 16), lambda i: (i, 0)),
            pl.BlockSpec((1, 128), lambda i: (0, 0)),
        ],
        out_shape=[
            jax.ShapeDtypeStruct((T, DIM), jnp.bfloat16),
            jax.ShapeDtypeStruct((nblk, sub, 128), jnp.int32),
            jax.ShapeDtypeStruct((nblk, sub, 128), jnp.int32),
            jax.ShapeDtypeStruct((nblk, sub, 128), jnp.int32),
            jax.ShapeDtypeStruct((nblk, sub, 128), jnp.int32),
            jax.ShapeDtypeStruct((T, 16), jnp.float32),
            jax.ShapeDtypeStruct((T, 16), jnp.float32),
            jax.ShapeDtypeStruct((1, 128), jnp.float32),
        ],
        scratch_shapes=[pltpu.VMEM((1, 128), jnp.float32)],
    )(xf, gate_w)


def _route_body(cnt_ref, e0_ref, e1_ref, r0_ref, r1_ref,
                p0_ref, p1_ref, tmap_ref, act_ref):
    counts = cnt_ref[0:1, 0:E].astype(jnp.int32)
    ntiles = (counts + (TILE - 1)) // TILE
    lane8 = lax.broadcasted_iota(jnp.int32, (1, E), 1)
    cumtiles = jnp.zeros((1, E), jnp.int32)
    for ep in range(E):
        nt_e = lax.slice_in_dim(ntiles, ep, ep + 1, axis=1)
        cumtiles = cumtiles + jnp.where(lane8 >= ep, nt_e, 0)
    off_rows = (cumtiles - ntiles) * TILE
    used = lax.slice_in_dim(cumtiles, E - 1, E, axis=1)
    laste = jnp.max(jnp.where(counts > 0, lane8, -1), axis=1, keepdims=True)

    j128 = lax.broadcasted_iota(jnp.int32, (1, 128), 1)
    texp = jnp.zeros((1, 128), jnp.int32)
    for ep in range(E):
        ct_e = lax.slice_in_dim(cumtiles, ep, ep + 1, axis=1)
        texp = texp + (j128 >= ct_e).astype(jnp.int32)
    tmap = jnp.where(j128 >= used, laste, texp)
    tmap_ref[...] = tmap
    act_ref[...] = (j128 < used).astype(jnp.int32)

    e0 = e0_ref[...]
    e1 = e1_ref[...]
    p0 = r0_ref[...]
    p1 = r1_ref[...]
    for ep in range(E):
        off_e = lax.slice_in_dim(off_rows, ep, ep + 1, axis=1)
        p0 = p0 + jnp.where(e0 == ep, off_e, 0)
        p1 = p1 + jnp.where(e1 == ep, off_e, 0)
    p0_ref[...] = p0
    p1_ref[...] = p1


def _route(cnt, e0, e1, r0, r1):
    nblk = T // BLK
    sub = BLK // 128
    return pl.pallas_call(
        _route_body,
        grid=(nblk,),
        in_specs=[
            pl.BlockSpec((1, 128), lambda i: (0, 0)),
            pl.BlockSpec((1, sub, 128), lambda i: (i, 0, 0)),
            pl.BlockSpec((1, sub, 128), lambda i: (i, 0, 0)),
            pl.BlockSpec((1, sub, 128), lambda i: (i, 0, 0)),
            pl.BlockSpec((1, sub, 128), lambda i: (i, 0, 0)),
        ],
        out_specs=[
            pl.BlockSpec((1, sub, 128), lambda i: (i, 0, 0)),
            pl.BlockSpec((1, sub, 128), lambda i: (i, 0, 0)),
            pl.BlockSpec((1, 128), lambda i: (0, 0)),
            pl.BlockSpec((1, 128), lambda i: (0, 0)),
        ],
        out_shape=[
            jax.ShapeDtypeStruct((nblk, sub, 128), jnp.int32),
            jax.ShapeDtypeStruct((nblk, sub, 128), jnp.int32),
            jax.ShapeDtypeStruct((1, 128), jnp.int32),
            jax.ShapeDtypeStruct((1, 128), jnp.int32),
        ],
    )(cnt, e0, e1, r0, r1)


def _sc_scatter(p0f, p1f):
    mesh = plsc.VectorSubcoreMesh(core_axis_name="c", subcore_axis_name="s")

    @functools.partial(
        pl.kernel, mesh=mesh,
        compiler_params=pltpu.CompilerParams(needs_layout_passes=False),
        out_type=jax.ShapeDtypeStruct((R_ROWS,), jnp.int32),
        scratch_types=[
            pltpu.VMEM((R_ROWS,), jnp.int32),
            pltpu.VMEM((T,), jnp.int32),
            pltpu.VMEM((T,), jnp.int32),
        ],
    )
    def k(p0_hbm, p1_hbm, tok_hbm, tok_v, p0_v, p1_v):
        wid = lax.axis_index("s") * NC + lax.axis_index("c")

        @pl.when(wid == 0)
        def _():
            pltpu.sync_copy(p0_hbm, p0_v)
            pltpu.sync_copy(p1_hbm, p1_v)
            iota = lax.iota(jnp.int32, L)
            zeros = jnp.zeros((L,), jnp.int32)

            def init(j, _):
                tok_v[pl.ds(j * L, L)] = zeros
                return 0

            lax.fori_loop(0, R_ROWS // L, init, 0, unroll=False)

            def scat(j, _):
                toks = iota + j * L
                idx0 = p0_v[pl.ds(j * L, L)]
                plsc.store_scatter(tok_v, [idx0], toks)
                idx1 = p1_v[pl.ds(j * L, L)]
                plsc.store_scatter(tok_v, [idx1], toks)
                return 0

            lax.fori_loop(0, T // L, scat, 0, unroll=False)
            pltpu.sync_copy(tok_v, tok_hbm)

    return k(p0f, p1f)


GCH = 96


def _sc_gather(xi32, tok_src):
    mesh = plsc.VectorSubcoreMesh(core_axis_name="c", subcore_axis_name="s")
    rows_per_w = R_ROWS // NW
    nch = rows_per_w // GCH

    @functools.partial(
        pl.kernel, mesh=mesh,
        compiler_params=pltpu.CompilerParams(needs_layout_passes=False),
        out_type=jax.ShapeDtypeStruct((R_ROWS, DIM // 2), jnp.int32),
        scratch_types=[
            pltpu.VMEM((GCH,), jnp.int32),
            pltpu.VMEM((GCH,), jnp.int32),
            pltpu.VMEM((GCH, DIM // 2), jnp.int32),
            pltpu.VMEM((GCH, DIM // 2), jnp.int32),
            pltpu.SemaphoreType.DMA,
            pltpu.SemaphoreType.DMA,
            pltpu.SemaphoreType.DMA,
            pltpu.SemaphoreType.DMA,
        ],
    )
    def k(x_hbm, tok_hbm, out_hbm, idx_v0, idx_v1, rows_v0, rows_v1,
          g0, g1, o0, o1):
        wid = lax.axis_index("s") * NC + lax.axis_index("c")
        base = wid * rows_per_w
        idx_v = (idx_v0, idx_v1)
        rows_v = (rows_v0, rows_v1)
        gsem = (g0, g1)
        osem = (o0, o1)
        gathers = [None] * nch
        outs = [None] * nch

        pltpu.sync_copy(tok_hbm.at[pl.ds(base, GCH)], idx_v0)
        gathers[0] = pltpu.async_copy(x_hbm.at[idx_v0], rows_v0, g0)
        for c in range(nch):
            b = c % 2
            nb = (c + 1) % 2
            if c + 1 < nch:
                if c >= 1:
                    outs[c - 1].wait()
                pltpu.sync_copy(tok_hbm.at[pl.ds(base + (c + 1) * GCH, GCH)],
                                idx_v[nb])
                gathers[c + 1] = pltpu.async_copy(x_hbm.at[idx_v[nb]],
                                                  rows_v[nb], gsem[nb])
            gathers[c].wait()
            outs[c] = pltpu.async_copy(
                rows_v[b], out_hbm.at[pl.ds(base + c * GCH, GCH)], osem[b])
        outs[nch - 2].wait()
        outs[nch - 1].wait()

    return k(xi32, tok_src)


def _ffn_math(xb, w1, w2, w3):
    h1 = lax.dot_general(xb, w1, (((1,), (1,)), ((), ())),
                         preferred_element_type=jnp.float32)
    h3 = lax.dot_general(xb, w3, (((1,), (1,)), ((), ())),
                         preferred_element_type=jnp.float32)
    g = (h1 * jax.nn.sigmoid(h1)) * h3
    return lax.dot_general(g.astype(jnp.bfloat16), w2, (((1,), (1,)), ((), ())),
                           preferred_element_type=jnp.float32)


HB = 688
NH = HIDDEN // HB


def _ffn_routed_body(tmap_ref, act_ref, x_ref, w1_ref, w2_ref, w3_ref, out_ref):
    t = pl.program_id(0)
    h = pl.program_id(1)

    @pl.when(act_ref[t] == 1)
    def _():
        xb = x_ref[...]
        h1 = lax.dot_general(xb, w1_ref[0], (((1,), (1,)), ((), ())),
                             preferred_element_type=jnp.float32)
        h3 = lax.dot_general(xb, w3_ref[0], (((1,), (1,)), ((), ())),
                             preferred_element_type=jnp.float32)
        g = (h1 * jax.nn.sigmoid(h1)) * h3
        part = lax.dot_general(g.astype(jnp.bfloat16), w2_ref[0],
                               (((1,), (0,)), ((), ())),
                               preferred_element_type=jnp.float32)

        @pl.when(h == 0)
        def _():
            out_ref[...] = part

        @pl.when(h != 0)
        def _():
            out_ref[...] += part


def _ffn_routed(Xg, W1b, W2tb, W3b, tmap, act):
    grid_spec = pltpu.PrefetchScalarGridSpec(
        num_scalar_prefetch=2,
        grid=(R_TILES, NH),
        in_specs=[
            pl.BlockSpec((TILE, DIM), lambda t, h, tm, ac: (t, 0)),
            pl.BlockSpec((1, HB, DIM), lambda t, h, tm, ac: (tm[t], h, 0)),
            pl.BlockSpec((1, HB, DIM), lambda t, h, tm, ac: (tm[t], h, 0)),
            pl.BlockSpec((1, HB, DIM), lambda t, h, tm, ac: (tm[t], h, 0)),
        ],
        out_specs=pl.BlockSpec((TILE, DIM), lambda t, h, tm, ac: (t, 0)),
    )
    return pl.pallas_call(
        _ffn_routed_body,
        grid_spec=grid_spec,
        out_shape=jax.ShapeDtypeStruct((R_ROWS, DIM), jnp.float32),
    )(tmap, act, Xg, W1b, W2tb, W3b)


def _ffn_shared_body(x_ref, w1_ref, w2_ref, w3_ref, out_ref):
    out_ref[...] = _ffn_math(x_ref[...], w1_ref[...], w2_ref[...], w3_ref[...])


def _ffn_shared(xb16f, Sw1b, Sw2b, Sw3b):
    return pl.pallas_call(
        _ffn_shared_body,
        grid=(S_TILES,),
        in_specs=[
            pl.BlockSpec((TILE, DIM), lambda t: (t, 0)),
            pl.BlockSpec((HIDDEN, DIM), lambda t: (0, 0)),
            pl.BlockSpec((DIM, HIDDEN), lambda t: (0, 0)),
            pl.BlockSpec((HIDDEN, DIM), lambda t: (0, 0)),
        ],
        out_specs=pl.BlockSpec((TILE, DIM), lambda t: (t, 0)),
        out_shape=jax.ShapeDtypeStruct((T, DIM), jnp.float32),
    )(xb16f, Sw1b, Sw2b, Sw3b)


CCH = 16


def _sc_combine(Yr, Ys, p0f, p1f, w0r, w1r):
    mesh = plsc.VectorSubcoreMesh(core_axis_name="c", subcore_axis_name="s")
    tok_per_w = T // NW

    @functools.partial(
        pl.kernel, mesh=mesh,
        compiler_params=pltpu.CompilerParams(needs_layout_passes=False),
        out_type=jax.ShapeDtypeStruct((T, DIM), jnp.float32),
        scratch_types=[
            pltpu.VMEM((tok_per_w,), jnp.int32),
            pltpu.VMEM((tok_per_w,), jnp.int32),
            pltpu.VMEM((tok_per_w, 16), jnp.float32),
            pltpu.VMEM((tok_per_w, 16), jnp.float32),
            pltpu.VMEM((CCH, DIM), jnp.float32),
            pltpu.VMEM((CCH, DIM), jnp.float32),
            pltpu.VMEM((CCH, DIM), jnp.float32),
            pltpu.VMEM((CCH, DIM), jnp.float32),
            pltpu.SemaphoreType.DMA,
            pltpu.SemaphoreType.DMA,
        ],
    )
    def k(yr_hbm, ys_hbm, p0_hbm, p1_hbm, w0_hbm, w1_hbm, y_hbm,
          p0_v, p1_v, w0_v, w1_v, r0_v, r1_v, rs_v, out_v, sem0, sem1):
        wid = lax.axis_index("s") * NC + lax.axis_index("c")
        base = wid * tok_per_w
        pltpu.sync_copy(p0_hbm.at[pl.ds(base, tok_per_w)], p0_v)
        pltpu.sync_copy(p1_hbm.at[pl.ds(base, tok_per_w)], p1_v)
        pltpu.sync_copy(w0_hbm.at[pl.ds(base, tok_per_w)], w0_v)
        pltpu.sync_copy(w1_hbm.at[pl.ds(base, tok_per_w)], w1_v)

        for c in range(tok_per_w // CCH):
            tbase = base + c * CCH
            idx0 = p0_v[pl.ds(c * CCH, CCH)]
            cp0 = pltpu.async_copy(yr_hbm.at[idx0], r0_v, sem0)
            idx1 = p1_v[pl.ds(c * CCH, CCH)]
            cp1 = pltpu.async_copy(yr_hbm.at[idx1], r1_v, sem1)
            pltpu.sync_copy(ys_hbm.at[pl.ds(tbase, CCH)], rs_v)
            cp0.wait()
            cp1.wait()
            for i in range(CCH):
                s0 = w0_v[c * CCH + i, :]
                s1 = w1_v[c * CCH + i, :]

                def feat(j, _):
                    sl = pl.ds(j * L, L)
                    out_v[i, sl] = (s0 * r0_v[i, sl] + s1 * r1_v[i, sl]
                                    + rs_v[i, sl])
                    return 0

                lax.fori_loop(0, DIM // L, feat, 0, unroll=4)
            pltpu.sync_copy(out_v, y_hbm.at[pl.ds(tbase, CCH)])

    return k(Yr, Ys, p0f, p1f, w0r, w1r)


@jax.jit
def kernel(x, gate_w, W1, W2, W3, Sw1, Sw2, Sw3):
    b, s, d = x.shape
    xf = x.reshape(-1, d)
    xb16, e0, e1, r0, r1, w0r, w1r, cnt = _gate(xf, gate_w)
    xb16f = xb16
    Ys = _ffn_shared(xb16f, Sw1.astype(jnp.bfloat16),
                     Sw2.astype(jnp.bfloat16), Sw3.astype(jnp.bfloat16))
    p0, p1, tmap, act = _route(cnt, e0, e1, r0, r1)
    p0f = p0.reshape(T)
    p1f = p1.reshape(T)
    tok_src = _sc_scatter(p0f, p1f)
    xi32 = lax.bitcast_convert_type(
        xb16.reshape(T, DIM // 2, 2), jnp.int32)
    Xg32 = _sc_gather(xi32, tok_src)
    Xg = lax.bitcast_convert_type(Xg32, jnp.bfloat16).reshape(R_ROWS, DIM)
    W2tb = jnp.swapaxes(W2, 1, 2).astype(jnp.bfloat16)
    Yr = _ffn_routed(Xg, W1.astype(jnp.bfloat16), W2tb,
                     W3.astype(jnp.bfloat16),
                     tmap.reshape(128), act.reshape(128))
    y = _sc_combine(Yr, Ys, p0f, p1f, w0r, w1r)
    return y.reshape(b, s, d)


# speedup vs baseline: 1.0035x; 1.0035x over previous
import functools

import jax
import jax.numpy as jnp
from jax import lax
from jax.experimental import pallas as pl
from jax.experimental.pallas import tpu as pltpu
from jax.experimental.pallas import tpu_sc as plsc

DIM = 1024
HIDDEN = 2752
E = 8
T = 4096

BLK = 512
TILE = 512
R_TILES = T * 2 // TILE + E
R_ROWS = R_TILES * TILE
S_TILES = T // TILE

NC, NS, L = 2, 16, 16
NW = NC * NS


def _gate_body(x_ref, gw_ref, xb_ref, e0_ref, e1_ref, r0_ref, r1_ref,
               w0_ref, w1_ref, cnt_ref, carry):
    i = pl.program_id(0)

    @pl.when(i == 0)
    def _():
        carry[...] = jnp.zeros_like(carry)

    xb = x_ref[...]
    xb_ref[...] = xb.astype(jnp.bfloat16)
    logits = lax.dot_general(xb, gw_ref[...], (((1,), (1,)), ((), ())),
                             preferred_element_type=jnp.float32)
    m = jnp.max(logits, axis=1, keepdims=True)
    ex = jnp.exp(logits - m)
    scores = ex / jnp.sum(ex, axis=1, keepdims=True)
    lane = lax.broadcasted_iota(jnp.int32, scores.shape, 1)
    rank = jnp.zeros(scores.shape, jnp.int32)
    for ep in range(E):
        sc = lax.slice_in_dim(scores, ep, ep + 1, axis=1)
        beats = (sc > scores) | ((sc == scores) & (ep < lane))
        rank = rank + beats.astype(jnp.int32)
    is0 = rank == 0
    is1 = rank == 1
    e0 = jnp.sum(jnp.where(is0, lane, 0), axis=1, keepdims=True)
    e1 = jnp.sum(jnp.where(is1, lane, 0), axis=1, keepdims=True)
    w0 = jnp.sum(jnp.where(is0, scores, 0.0), axis=1, keepdims=True)
    w1 = jnp.sum(jnp.where(is1, scores, 0.0), axis=1, keepdims=True)

    oh = (is0 | is1).astype(jnp.float32)
    tri = (lax.broadcasted_iota(jnp.int32, (BLK, BLK), 0)
           > lax.broadcasted_iota(jnp.int32, (BLK, BLK), 1)).astype(jnp.float32)
    excl = lax.dot_general(tri, oh, (((1,), (0,)), ((), ())),
                           preferred_element_type=jnp.float32)
    cbase = carry[0:1, 0:E]
    r_both = excl + cbase
    r0 = jnp.sum(jnp.where(is0, r_both, 0.0), axis=1, keepdims=True)
    r1 = jnp.sum(jnp.where(is1, r_both, 0.0), axis=1, keepdims=True)

    new_carry = cbase + jnp.sum(oh, axis=0, keepdims=True)
    carry[...] = jnp.pad(new_carry, ((0, 0), (0, 128 - E)))
    cnt_ref[...] = carry[...]

    e0_ref[...] = e0.astype(jnp.int32).reshape(1, BLK // 128, 128)
    e1_ref[...] = e1.astype(jnp.int32).reshape(1, BLK // 128, 128)
    r0_ref[...] = r0.astype(jnp.int32).reshape(1, BLK // 128, 128)
    r1_ref[...] = r1.astype(jnp.int32).reshape(1, BLK // 128, 128)
    w0_ref[...] = jnp.broadcast_to(w0, (BLK, 16))
    w1_ref[...] = jnp.broadcast_to(w1, (BLK, 16))


def _gate(xf, gate_w):
    nblk = T // BLK
    sub = BLK // 128
    return pl.pallas_call(
        _gate_body,
        grid=(nblk,),
        in_specs=[
            pl.BlockSpec((BLK, DIM), lambda i: (i, 0)),
            pl.BlockSpec((E, DIM), lambda i: (0, 0)),
        ],
        out_specs=[
            pl.BlockSpec((BLK, DIM), lambda i: (i, 0)),
            pl.BlockSpec((1, sub, 128), lambda i: (i, 0, 0)),
            pl.BlockSpec((1, sub, 128), lambda i: (i, 0, 0)),
            pl.BlockSpec((1, sub, 128), lambda i: (i, 0, 0)),
            pl.BlockSpec((1, sub, 128), lambda i: (i, 0, 0)),
            pl.BlockSpec((BLK, 16), lambda i: (i, 0)),
            pl.BlockSpec((BLK, 16), lambda i: (i, 0)),
            pl.BlockSpec((1, 128), lambda i: (0, 0)),
        ],
        out_shape=[
            jax.ShapeDtypeStruct((T, DIM), jnp.bfloat16),
            jax.ShapeDtypeStruct((nblk, sub, 128), jnp.int32),
            jax.ShapeDtypeStruct((nblk, sub, 128), jnp.int32),
            jax.ShapeDtypeStruct((nblk, sub, 128), jnp.int32),
            jax.ShapeDtypeStruct((nblk, sub, 128), jnp.int32),
            jax.ShapeDtypeStruct((T, 16), jnp.float32),
            jax.ShapeDtypeStruct((T, 16), jnp.float32),
            jax.ShapeDtypeStruct((1, 128), jnp.float32),
        ],
        scratch_shapes=[pltpu.VMEM((1, 128), jnp.float32)],
    )(xf, gate_w)


def _route_body(cnt_ref, e0_ref, e1_ref, r0_ref, r1_ref,
                p0_ref, p1_ref, tmap_ref, act_ref):
    counts = cnt_ref[0:1, 0:E].astype(jnp.int32)
    ntiles = (counts + (TILE - 1)) // TILE
    lane8 = lax.broadcasted_iota(jnp.int32, (1, E), 1)
    cumtiles = jnp.zeros((1, E), jnp.int32)
    for ep in range(E):
        nt_e = lax.slice_in_dim(ntiles, ep, ep + 1, axis=1)
        cumtiles = cumtiles + jnp.where(lane8 >= ep, nt_e, 0)
    off_rows = (cumtiles - ntiles) * TILE
    used = lax.slice_in_dim(cumtiles, E - 1, E, axis=1)
    laste = jnp.max(jnp.where(counts > 0, lane8, -1), axis=1, keepdims=True)

    j128 = lax.broadcasted_iota(jnp.int32, (1, 128), 1)
    texp = jnp.zeros((1, 128), jnp.int32)
    for ep in range(E):
        ct_e = lax.slice_in_dim(cumtiles, ep, ep + 1, axis=1)
        texp = texp + (j128 >= ct_e).astype(jnp.int32)
    tmap = jnp.where(j128 >= used, laste, texp)
    tmap_ref[...] = tmap
    act_ref[...] = (j128 < used).astype(jnp.int32)

    e0 = e0_ref[...]
    e1 = e1_ref[...]
    p0 = r0_ref[...]
    p1 = r1_ref[...]
    for ep in range(E):
        off_e = lax.slice_in_dim(off_rows, ep, ep + 1, axis=1)
        p0 = p0 + jnp.where(e0 == ep, off_e, 0)
        p1 = p1 + jnp.where(e1 == ep, off_e, 0)
    p0_ref[...] = p0
    p1_ref[...] = p1


def _route(cnt, e0, e1, r0, r1):
    nblk = T // BLK
    sub = BLK // 128
    return pl.pallas_call(
        _route_body,
        grid=(nblk,),
        in_specs=[
            pl.BlockSpec((1, 128), lambda i: (0, 0)),
            pl.BlockSpec((1, sub, 128), lambda i: (i, 0, 0)),
            pl.BlockSpec((1, sub, 128), lambda i: (i, 0, 0)),
            pl.BlockSpec((1, sub, 128), lambda i: (i, 0, 0)),
            pl.BlockSpec((1, sub, 128), lambda i: (i, 0, 0)),
        ],
        out_specs=[
            pl.BlockSpec((1, sub, 128), lambda i: (i, 0, 0)),
            pl.BlockSpec((1, sub, 128), lambda i: (i, 0, 0)),
            pl.BlockSpec((1, 128), lambda i: (0, 0)),
            pl.BlockSpec((1, 128), lambda i: (0, 0)),
        ],
        out_shape=[
            jax.ShapeDtypeStruct((nblk, sub, 128), jnp.int32),
            jax.ShapeDtypeStruct((nblk, sub, 128), jnp.int32),
            jax.ShapeDtypeStruct((1, 128), jnp.int32),
            jax.ShapeDtypeStruct((1, 128), jnp.int32),
        ],
    )(cnt, e0, e1, r0, r1)


def _sc_scatter(p0f, p1f):
    mesh = plsc.VectorSubcoreMesh(core_axis_name="c", subcore_axis_name="s")

    @functools.partial(
        pl.kernel, mesh=mesh,
        compiler_params=pltpu.CompilerParams(needs_layout_passes=False),
        out_type=jax.ShapeDtypeStruct((R_ROWS,), jnp.int32),
        scratch_types=[
            pltpu.VMEM((R_ROWS,), jnp.int32),
            pltpu.VMEM((T,), jnp.int32),
            pltpu.VMEM((T,), jnp.int32),
        ],
    )
    def k(p0_hbm, p1_hbm, tok_hbm, tok_v, p0_v, p1_v):
        wid = lax.axis_index("s") * NC + lax.axis_index("c")

        @pl.when(wid == 0)
        def _():
            pltpu.sync_copy(p0_hbm, p0_v)
            pltpu.sync_copy(p1_hbm, p1_v)
            iota = lax.iota(jnp.int32, L)
            zeros = jnp.zeros((L,), jnp.int32)

            def init(j, _):
                tok_v[pl.ds(j * L, L)] = zeros
                return 0

            lax.fori_loop(0, R_ROWS // L, init, 0, unroll=False)

            def scat(j, _):
                toks = iota + j * L
                idx0 = p0_v[pl.ds(j * L, L)]
                plsc.store_scatter(tok_v, [idx0], toks)
                idx1 = p1_v[pl.ds(j * L, L)]
                plsc.store_scatter(tok_v, [idx1], toks)
                return 0

            lax.fori_loop(0, T // L, scat, 0, unroll=False)
            pltpu.sync_copy(tok_v, tok_hbm)

    return k(p0f, p1f)


GCH = 96


def _sc_gather(xi32, tok_src):
    mesh = plsc.VectorSubcoreMesh(core_axis_name="c", subcore_axis_name="s")
    rows_per_w = R_ROWS // NW
    nch = rows_per_w // GCH

    @functools.partial(
        pl.kernel, mesh=mesh,
        compiler_params=pltpu.CompilerParams(needs_layout_passes=False),
        out_type=jax.ShapeDtypeStruct((R_ROWS, DIM // 2), jnp.int32),
        scratch_types=[
            pltpu.VMEM((GCH,), jnp.int32),
            pltpu.VMEM((GCH,), jnp.int32),
            pltpu.VMEM((GCH, DIM // 2), jnp.int32),
            pltpu.VMEM((GCH, DIM // 2), jnp.int32),
            pltpu.SemaphoreType.DMA,
            pltpu.SemaphoreType.DMA,
            pltpu.SemaphoreType.DMA,
            pltpu.SemaphoreType.DMA,
        ],
    )
    def k(x_hbm, tok_hbm, out_hbm, idx_v0, idx_v1, rows_v0, rows_v1,
          g0, g1, o0, o1):
        wid = lax.axis_index("s") * NC + lax.axis_index("c")
        base = wid * rows_per_w
        idx_v = (idx_v0, idx_v1)
        rows_v = (rows_v0, rows_v1)
        gsem = (g0, g1)
        osem = (o0, o1)
        gathers = [None] * nch
        outs = [None] * nch

        pltpu.sync_copy(tok_hbm.at[pl.ds(base, GCH)], idx_v0)
        gathers[0] = pltpu.async_copy(x_hbm.at[idx_v0], rows_v0, g0)
        for c in range(nch):
            b = c % 2
            nb = (c + 1) % 2
            if c + 1 < nch:
                if c >= 1:
                    outs[c - 1].wait()
                pltpu.sync_copy(tok_hbm.at[pl.ds(base + (c + 1) * GCH, GCH)],
                                idx_v[nb])
                gathers[c + 1] = pltpu.async_copy(x_hbm.at[idx_v[nb]],
                                                  rows_v[nb], gsem[nb])
            gathers[c].wait()
            outs[c] = pltpu.async_copy(
                rows_v[b], out_hbm.at[pl.ds(base + c * GCH, GCH)], osem[b])
        outs[nch - 2].wait()
        outs[nch - 1].wait()

    return k(xi32, tok_src)


def _ffn_math(xb, w1, w2, w3):
    h1 = lax.dot_general(xb, w1, (((1,), (1,)), ((), ())),
                         preferred_element_type=jnp.float32)
    h3 = lax.dot_general(xb, w3, (((1,), (1,)), ((), ())),
                         preferred_element_type=jnp.float32)
    g = (h1 * jax.nn.sigmoid(h1)) * h3
    return lax.dot_general(g.astype(jnp.bfloat16), w2, (((1,), (1,)), ((), ())),
                           preferred_element_type=jnp.float32)


def _ffn_routed_body(tmap_ref, act_ref, x_ref, w1_ref, w2_ref, w3_ref, out_ref):
    t = pl.program_id(0)

    @pl.when(act_ref[t] == 1)
    def _():
        out_ref[...] = _ffn_math(x_ref[...], w1_ref[0], w2_ref[0], w3_ref[0])


def _ffn_routed(Xg, W1b, W2b, W3b, tmap, act):
    grid_spec = pltpu.PrefetchScalarGridSpec(
        num_scalar_prefetch=2,
        grid=(R_TILES,),
        in_specs=[
            pl.BlockSpec((TILE, DIM), lambda t, tm, ac: (t, 0)),
            pl.BlockSpec((1, HIDDEN, DIM), lambda t, tm, ac: (tm[t], 0, 0)),
            pl.BlockSpec((1, DIM, HIDDEN), lambda t, tm, ac: (tm[t], 0, 0)),
            pl.BlockSpec((1, HIDDEN, DIM), lambda t, tm, ac: (tm[t], 0, 0)),
        ],
        out_specs=pl.BlockSpec((TILE, DIM), lambda t, tm, ac: (t, 0)),
    )
    return pl.pallas_call(
        _ffn_routed_body,
        grid_spec=grid_spec,
        out_shape=jax.ShapeDtypeStruct((R_ROWS, DIM), jnp.float32),
    )(tmap, act, Xg, W1b, W2b, W3b)


def _ffn_shared_body(x_ref, w1_ref, w2_ref, w3_ref, out_ref):
    out_ref[...] = _ffn_math(x_ref[...], w1_ref[...], w2_ref[...], w3_ref[...])


def _ffn_shared(xb16f, Sw1b, Sw2b, Sw3b):
    return pl.pallas_call(
        _ffn_shared_body,
        grid=(S_TILES,),
        in_specs=[
            pl.BlockSpec((TILE, DIM), lambda t: (t, 0)),
            pl.BlockSpec((HIDDEN, DIM), lambda t: (0, 0)),
            pl.BlockSpec((DIM, HIDDEN), lambda t: (0, 0)),
            pl.BlockSpec((HIDDEN, DIM), lambda t: (0, 0)),
        ],
        out_specs=pl.BlockSpec((TILE, DIM), lambda t: (t, 0)),
        out_shape=jax.ShapeDtypeStruct((T, DIM), jnp.float32),
    )(xb16f, Sw1b, Sw2b, Sw3b)


CCH = 16


def _sc_combine(Yr, Ys, p0f, p1f, w0r, w1r):
    mesh = plsc.VectorSubcoreMesh(core_axis_name="c", subcore_axis_name="s")
    tok_per_w = T // NW

    @functools.partial(
        pl.kernel, mesh=mesh,
        compiler_params=pltpu.CompilerParams(needs_layout_passes=False),
        out_type=jax.ShapeDtypeStruct((T, DIM), jnp.float32),
        scratch_types=[
            pltpu.VMEM((tok_per_w,), jnp.int32),
            pltpu.VMEM((tok_per_w,), jnp.int32),
            pltpu.VMEM((tok_per_w, 16), jnp.float32),
            pltpu.VMEM((tok_per_w, 16), jnp.float32),
            pltpu.VMEM((CCH, DIM), jnp.float32),
            pltpu.VMEM((CCH, DIM), jnp.float32),
            pltpu.VMEM((CCH, DIM), jnp.float32),
            pltpu.VMEM((CCH, DIM), jnp.float32),
            pltpu.SemaphoreType.DMA,
            pltpu.SemaphoreType.DMA,
        ],
    )
    def k(yr_hbm, ys_hbm, p0_hbm, p1_hbm, w0_hbm, w1_hbm, y_hbm,
          p0_v, p1_v, w0_v, w1_v, r0_v, r1_v, rs_v, out_v, sem0, sem1):
        wid = lax.axis_index("s") * NC + lax.axis_index("c")
        base = wid * tok_per_w
        pltpu.sync_copy(p0_hbm.at[pl.ds(base, tok_per_w)], p0_v)
        pltpu.sync_copy(p1_hbm.at[pl.ds(base, tok_per_w)], p1_v)
        pltpu.sync_copy(w0_hbm.at[pl.ds(base, tok_per_w)], w0_v)
        pltpu.sync_copy(w1_hbm.at[pl.ds(base, tok_per_w)], w1_v)

        for c in range(tok_per_w // CCH):
            tbase = base + c * CCH
            idx0 = p0_v[pl.ds(c * CCH, CCH)]
            cp0 = pltpu.async_copy(yr_hbm.at[idx0], r0_v, sem0)
            idx1 = p1_v[pl.ds(c * CCH, CCH)]
            cp1 = pltpu.async_copy(yr_hbm.at[idx1], r1_v, sem1)
            pltpu.sync_copy(ys_hbm.at[pl.ds(tbase, CCH)], rs_v)
            cp0.wait()
            cp1.wait()
            for i in range(CCH):
                s0 = w0_v[c * CCH + i, :]
                s1 = w1_v[c * CCH + i, :]

                def feat(j, _):
                    sl = pl.ds(j * L, L)
                    out_v[i, sl] = (s0 * r0_v[i, sl] + s1 * r1_v[i, sl]
                                    + rs_v[i, sl])
                    return 0

                lax.fori_loop(0, DIM // L, feat, 0, unroll=4)
            pltpu.sync_copy(out_v, y_hbm.at[pl.ds(tbase, CCH)])

    return k(Yr, Ys, p0f, p1f, w0r, w1r)


@jax.jit
def kernel(x, gate_w, W1, W2, W3, Sw1, Sw2, Sw3):
    b, s, d = x.shape
    xf = x.reshape(-1, d)
    xb16, e0, e1, r0, r1, w0r, w1r, cnt = _gate(xf, gate_w)
    xb16f = xb16
    Ys = _ffn_shared(xb16f, Sw1.astype(jnp.bfloat16),
                     Sw2.astype(jnp.bfloat16), Sw3.astype(jnp.bfloat16))
    p0, p1, tmap, act = _route(cnt, e0, e1, r0, r1)
    p0f = p0.reshape(T)
    p1f = p1.reshape(T)
    tok_src = _sc_scatter(p0f, p1f)
    xi32 = lax.bitcast_convert_type(
        xb16.reshape(T, DIM // 2, 2), jnp.int32)
    Xg32 = _sc_gather(xi32, tok_src)
    Xg = lax.bitcast_convert_type(Xg32, jnp.bfloat16).reshape(R_ROWS, DIM)
    Yr = _ffn_routed(Xg, W1.astype(jnp.bfloat16), W2.astype(jnp.bfloat16),
                     W3.astype(jnp.bfloat16),
                     tmap.reshape(128), act.reshape(128))
    y = _sc_combine(Yr, Ys, p0f, p1f, w0r, w1r)
    return y.reshape(b, s, d)


# speedup vs baseline: 1.1361x; 1.1321x over previous
import functools

import jax
import jax.numpy as jnp
from jax import lax
from jax.experimental import pallas as pl
from jax.experimental.pallas import tpu as pltpu
from jax.experimental.pallas import tpu_sc as plsc

DIM = 1024
HIDDEN = 2752
E = 8
T = 4096

BLK = 512
TILE = 256
R_TILES = T * 2 // TILE + E
R_ROWS = R_TILES * TILE
S_TILES = T // TILE

NC, NS, L = 2, 16, 16
NW = NC * NS


def _gate_body(x_ref, gw_ref, xb_ref, e0_ref, e1_ref, r0_ref, r1_ref,
               w0_ref, w1_ref, cnt_ref, carry):
    i = pl.program_id(0)

    @pl.when(i == 0)
    def _():
        carry[...] = jnp.zeros_like(carry)

    xb = x_ref[...]
    xb_ref[...] = xb.astype(jnp.bfloat16)
    logits = lax.dot_general(xb, gw_ref[...], (((1,), (1,)), ((), ())),
                             preferred_element_type=jnp.float32)
    m = jnp.max(logits, axis=1, keepdims=True)
    ex = jnp.exp(logits - m)
    scores = ex / jnp.sum(ex, axis=1, keepdims=True)
    lane = lax.broadcasted_iota(jnp.int32, scores.shape, 1)
    rank = jnp.zeros(scores.shape, jnp.int32)
    for ep in range(E):
        sc = lax.slice_in_dim(scores, ep, ep + 1, axis=1)
        beats = (sc > scores) | ((sc == scores) & (ep < lane))
        rank = rank + beats.astype(jnp.int32)
    is0 = rank == 0
    is1 = rank == 1
    e0 = jnp.sum(jnp.where(is0, lane, 0), axis=1, keepdims=True)
    e1 = jnp.sum(jnp.where(is1, lane, 0), axis=1, keepdims=True)
    w0 = jnp.sum(jnp.where(is0, scores, 0.0), axis=1, keepdims=True)
    w1 = jnp.sum(jnp.where(is1, scores, 0.0), axis=1, keepdims=True)

    oh = (is0 | is1).astype(jnp.float32)
    tri = (lax.broadcasted_iota(jnp.int32, (BLK, BLK), 0)
           > lax.broadcasted_iota(jnp.int32, (BLK, BLK), 1)).astype(jnp.float32)
    excl = lax.dot_general(tri, oh, (((1,), (0,)), ((), ())),
                           preferred_element_type=jnp.float32)
    cbase = carry[0:1, 0:E]
    r_both = excl + cbase
    r0 = jnp.sum(jnp.where(is0, r_both, 0.0), axis=1, keepdims=True)
    r1 = jnp.sum(jnp.where(is1, r_both, 0.0), axis=1, keepdims=True)

    new_carry = cbase + jnp.sum(oh, axis=0, keepdims=True)
    carry[...] = jnp.pad(new_carry, ((0, 0), (0, 128 - E)))
    cnt_ref[...] = carry[...]

    e0_ref[...] = e0.astype(jnp.int32).reshape(1, BLK // 128, 128)
    e1_ref[...] = e1.astype(jnp.int32).reshape(1, BLK // 128, 128)
    r0_ref[...] = r0.astype(jnp.int32).reshape(1, BLK // 128, 128)
    r1_ref[...] = r1.astype(jnp.int32).reshape(1, BLK // 128, 128)
    w0_ref[...] = jnp.broadcast_to(w0, (BLK, 16))
    w1_ref[...] = jnp.broadcast_to(w1, (BLK, 16))


def _gate(xf, gate_w):
    nblk = T // BLK
    sub = BLK // 128
    return pl.pallas_call(
        _gate_body,
        grid=(nblk,),
        in_specs=[
            pl.BlockSpec((BLK, DIM), lambda i: (i, 0)),
            pl.BlockSpec((E, DIM), lambda i: (0, 0)),
        ],
        out_specs=[
            pl.BlockSpec((BLK, DIM), lambda i: (i, 0)),
            pl.BlockSpec((1, sub, 128), lambda i: (i, 0, 0)),
            pl.BlockSpec((1, sub, 128), lambda i: (i, 0, 0)),
            pl.BlockSpec((1, sub, 128), lambda i: (i, 0, 0)),
            pl.BlockSpec((1, sub, 128), lambda i: (i, 0, 0)),
            pl.BlockSpec((BLK, 16), lambda i: (i, 0)),
            pl.BlockSpec((BLK, 16), lambda i: (i, 0)),
            pl.BlockSpec((1, 128), lambda i: (0, 0)),
        ],
        out_shape=[
            jax.ShapeDtypeStruct((T, DIM), jnp.bfloat16),
            jax.ShapeDtypeStruct((nblk, sub, 128), jnp.int32),
            jax.ShapeDtypeStruct((nblk, sub, 128), jnp.int32),
            jax.ShapeDtypeStruct((nblk, sub, 128), jnp.int32),
            jax.ShapeDtypeStruct((nblk, sub, 128), jnp.int32),
            jax.ShapeDtypeStruct((T, 16), jnp.float32),
            jax.ShapeDtypeStruct((T, 16), jnp.float32),
            jax.ShapeDtypeStruct((1, 128), jnp.float32),
        ],
        scratch_shapes=[pltpu.VMEM((1, 128), jnp.float32)],
    )(xf, gate_w)


def _route_body(cnt_ref, e0_ref, e1_ref, r0_ref, r1_ref,
                p0_ref, p1_ref, tmap_ref, act_ref):
    counts = cnt_ref[0:1, 0:E].astype(jnp.int32)
    ntiles = (counts + (TILE - 1)) // TILE
    lane8 = lax.broadcasted_iota(jnp.int32, (1, E), 1)
    cumtiles = jnp.zeros((1, E), jnp.int32)
    for ep in range(E):
        nt_e = lax.slice_in_dim(ntiles, ep, ep + 1, axis=1)
        cumtiles = cumtiles + jnp.where(lane8 >= ep, nt_e, 0)
    off_rows = (cumtiles - ntiles) * TILE
    used = lax.slice_in_dim(cumtiles, E - 1, E, axis=1)
    laste = jnp.max(jnp.where(counts > 0, lane8, -1), axis=1, keepdims=True)

    j128 = lax.broadcasted_iota(jnp.int32, (1, 128), 1)
    texp = jnp.zeros((1, 128), jnp.int32)
    for ep in range(E):
        ct_e = lax.slice_in_dim(cumtiles, ep, ep + 1, axis=1)
        texp = texp + (j128 >= ct_e).astype(jnp.int32)
    tmap = jnp.where(j128 >= used, laste, texp)
    tmap_ref[...] = tmap
    act_ref[...] = (j128 < used).astype(jnp.int32)

    e0 = e0_ref[...]
    e1 = e1_ref[...]
    p0 = r0_ref[...]
    p1 = r1_ref[...]
    for ep in range(E):
        off_e = lax.slice_in_dim(off_rows, ep, ep + 1, axis=1)
        p0 = p0 + jnp.where(e0 == ep, off_e, 0)
        p1 = p1 + jnp.where(e1 == ep, off_e, 0)
    p0_ref[...] = p0
    p1_ref[...] = p1


def _route(cnt, e0, e1, r0, r1):
    nblk = T // BLK
    sub = BLK // 128
    return pl.pallas_call(
        _route_body,
        grid=(nblk,),
        in_specs=[
            pl.BlockSpec((1, 128), lambda i: (0, 0)),
            pl.BlockSpec((1, sub, 128), lambda i: (i, 0, 0)),
            pl.BlockSpec((1, sub, 128), lambda i: (i, 0, 0)),
            pl.BlockSpec((1, sub, 128), lambda i: (i, 0, 0)),
            pl.BlockSpec((1, sub, 128), lambda i: (i, 0, 0)),
        ],
        out_specs=[
            pl.BlockSpec((1, sub, 128), lambda i: (i, 0, 0)),
            pl.BlockSpec((1, sub, 128), lambda i: (i, 0, 0)),
            pl.BlockSpec((1, 128), lambda i: (0, 0)),
            pl.BlockSpec((1, 128), lambda i: (0, 0)),
        ],
        out_shape=[
            jax.ShapeDtypeStruct((nblk, sub, 128), jnp.int32),
            jax.ShapeDtypeStruct((nblk, sub, 128), jnp.int32),
            jax.ShapeDtypeStruct((1, 128), jnp.int32),
            jax.ShapeDtypeStruct((1, 128), jnp.int32),
        ],
    )(cnt, e0, e1, r0, r1)


def _sc_scatter(p0f, p1f):
    mesh = plsc.VectorSubcoreMesh(core_axis_name="c", subcore_axis_name="s")

    @functools.partial(
        pl.kernel, mesh=mesh,
        compiler_params=pltpu.CompilerParams(needs_layout_passes=False),
        out_type=jax.ShapeDtypeStruct((R_ROWS,), jnp.int32),
        scratch_types=[
            pltpu.VMEM((R_ROWS,), jnp.int32),
            pltpu.VMEM((T,), jnp.int32),
            pltpu.VMEM((T,), jnp.int32),
        ],
    )
    def k(p0_hbm, p1_hbm, tok_hbm, tok_v, p0_v, p1_v):
        wid = lax.axis_index("s") * NC + lax.axis_index("c")

        @pl.when(wid == 0)
        def _():
            pltpu.sync_copy(p0_hbm, p0_v)
            pltpu.sync_copy(p1_hbm, p1_v)
            iota = lax.iota(jnp.int32, L)
            zeros = jnp.zeros((L,), jnp.int32)

            def init(j, _):
                tok_v[pl.ds(j * L, L)] = zeros
                return 0

            lax.fori_loop(0, R_ROWS // L, init, 0, unroll=False)

            def scat(j, _):
                toks = iota + j * L
                idx0 = p0_v[pl.ds(j * L, L)]
                plsc.store_scatter(tok_v, [idx0], toks)
                idx1 = p1_v[pl.ds(j * L, L)]
                plsc.store_scatter(tok_v, [idx1], toks)
                return 0

            lax.fori_loop(0, T // L, scat, 0, unroll=False)
            pltpu.sync_copy(tok_v, tok_hbm)

    return k(p0f, p1f)


GCH = 80


def _sc_gather(xi32, tok_src):
    mesh = plsc.VectorSubcoreMesh(core_axis_name="c", subcore_axis_name="s")
    rows_per_w = R_ROWS // NW
    nch = rows_per_w // GCH

    @functools.partial(
        pl.kernel, mesh=mesh,
        compiler_params=pltpu.CompilerParams(needs_layout_passes=False),
        out_type=jax.ShapeDtypeStruct((R_ROWS, DIM // 2), jnp.int32),
        scratch_types=[
            pltpu.VMEM((GCH,), jnp.int32),
            pltpu.VMEM((GCH,), jnp.int32),
            pltpu.VMEM((GCH, DIM // 2), jnp.int32),
            pltpu.VMEM((GCH, DIM // 2), jnp.int32),
            pltpu.SemaphoreType.DMA,
            pltpu.SemaphoreType.DMA,
            pltpu.SemaphoreType.DMA,
            pltpu.SemaphoreType.DMA,
        ],
    )
    def k(x_hbm, tok_hbm, out_hbm, idx_v0, idx_v1, rows_v0, rows_v1,
          g0, g1, o0, o1):
        wid = lax.axis_index("s") * NC + lax.axis_index("c")
        base = wid * rows_per_w
        idx_v = (idx_v0, idx_v1)
        rows_v = (rows_v0, rows_v1)
        gsem = (g0, g1)
        osem = (o0, o1)
        gathers = [None] * nch
        outs = [None] * nch

        pltpu.sync_copy(tok_hbm.at[pl.ds(base, GCH)], idx_v0)
        gathers[0] = pltpu.async_copy(x_hbm.at[idx_v0], rows_v0, g0)
        for c in range(nch):
            b = c % 2
            nb = (c + 1) % 2
            if c + 1 < nch:
                if c >= 1:
                    outs[c - 1].wait()
                pltpu.sync_copy(tok_hbm.at[pl.ds(base + (c + 1) * GCH, GCH)],
                                idx_v[nb])
                gathers[c + 1] = pltpu.async_copy(x_hbm.at[idx_v[nb]],
                                                  rows_v[nb], gsem[nb])
            gathers[c].wait()
            outs[c] = pltpu.async_copy(
                rows_v[b], out_hbm.at[pl.ds(base + c * GCH, GCH)], osem[b])
        outs[nch - 2].wait()
        outs[nch - 1].wait()

    return k(xi32, tok_src)


def _ffn_math(xb, w1, w2, w3):
    h1 = lax.dot_general(xb, w1, (((1,), (1,)), ((), ())),
                         preferred_element_type=jnp.float32)
    h3 = lax.dot_general(xb, w3, (((1,), (1,)), ((), ())),
                         preferred_element_type=jnp.float32)
    g = (h1 * jax.nn.sigmoid(h1)) * h3
    return lax.dot_general(g.astype(jnp.bfloat16), w2, (((1,), (1,)), ((), ())),
                           preferred_element_type=jnp.float32)


def _ffn_routed_body(tmap_ref, act_ref, x_ref, w1_ref, w2_ref, w3_ref, out_ref):
    t = pl.program_id(0)

    @pl.when(act_ref[t] == 1)
    def _():
        out_ref[...] = _ffn_math(x_ref[...], w1_ref[0], w2_ref[0], w3_ref[0])


def _ffn_routed(Xg, W1b, W2b, W3b, tmap, act):
    grid_spec = pltpu.PrefetchScalarGridSpec(
        num_scalar_prefetch=2,
        grid=(R_TILES,),
        in_specs=[
            pl.BlockSpec((TILE, DIM), lambda t, tm, ac: (t, 0)),
            pl.BlockSpec((1, HIDDEN, DIM), lambda t, tm, ac: (tm[t], 0, 0)),
            pl.BlockSpec((1, DIM, HIDDEN), lambda t, tm, ac: (tm[t], 0, 0)),
            pl.BlockSpec((1, HIDDEN, DIM), lambda t, tm, ac: (tm[t], 0, 0)),
        ],
        out_specs=pl.BlockSpec((TILE, DIM), lambda t, tm, ac: (t, 0)),
    )
    return pl.pallas_call(
        _ffn_routed_body,
        grid_spec=grid_spec,
        out_shape=jax.ShapeDtypeStruct((R_ROWS, DIM), jnp.float32),
    )(tmap, act, Xg, W1b, W2b, W3b)


def _ffn_shared_body(x_ref, w1_ref, w2_ref, w3_ref, out_ref):
    out_ref[...] = _ffn_math(x_ref[...], w1_ref[...], w2_ref[...], w3_ref[...])


def _ffn_shared(xb16f, Sw1b, Sw2b, Sw3b):
    return pl.pallas_call(
        _ffn_shared_body,
        grid=(S_TILES,),
        in_specs=[
            pl.BlockSpec((TILE, DIM), lambda t: (t, 0)),
            pl.BlockSpec((HIDDEN, DIM), lambda t: (0, 0)),
            pl.BlockSpec((DIM, HIDDEN), lambda t: (0, 0)),
            pl.BlockSpec((HIDDEN, DIM), lambda t: (0, 0)),
        ],
        out_specs=pl.BlockSpec((TILE, DIM), lambda t: (t, 0)),
        out_shape=jax.ShapeDtypeStruct((T, DIM), jnp.float32),
    )(xb16f, Sw1b, Sw2b, Sw3b)


CCH = 16


def _sc_combine(Yr, Ys, p0f, p1f, w0r, w1r):
    mesh = plsc.VectorSubcoreMesh(core_axis_name="c", subcore_axis_name="s")
    tok_per_w = T // NW

    @functools.partial(
        pl.kernel, mesh=mesh,
        compiler_params=pltpu.CompilerParams(needs_layout_passes=False),
        out_type=jax.ShapeDtypeStruct((T, DIM), jnp.float32),
        scratch_types=[
            pltpu.VMEM((tok_per_w,), jnp.int32),
            pltpu.VMEM((tok_per_w,), jnp.int32),
            pltpu.VMEM((tok_per_w, 16), jnp.float32),
            pltpu.VMEM((tok_per_w, 16), jnp.float32),
            pltpu.VMEM((CCH, DIM), jnp.float32),
            pltpu.VMEM((CCH, DIM), jnp.float32),
            pltpu.VMEM((CCH, DIM), jnp.float32),
            pltpu.VMEM((CCH, DIM), jnp.float32),
            pltpu.SemaphoreType.DMA,
            pltpu.SemaphoreType.DMA,
        ],
    )
    def k(yr_hbm, ys_hbm, p0_hbm, p1_hbm, w0_hbm, w1_hbm, y_hbm,
          p0_v, p1_v, w0_v, w1_v, r0_v, r1_v, rs_v, out_v, sem0, sem1):
        wid = lax.axis_index("s") * NC + lax.axis_index("c")
        base = wid * tok_per_w
        pltpu.sync_copy(p0_hbm.at[pl.ds(base, tok_per_w)], p0_v)
        pltpu.sync_copy(p1_hbm.at[pl.ds(base, tok_per_w)], p1_v)
        pltpu.sync_copy(w0_hbm.at[pl.ds(base, tok_per_w)], w0_v)
        pltpu.sync_copy(w1_hbm.at[pl.ds(base, tok_per_w)], w1_v)

        for c in range(tok_per_w // CCH):
            tbase = base + c * CCH
            idx0 = p0_v[pl.ds(c * CCH, CCH)]
            cp0 = pltpu.async_copy(yr_hbm.at[idx0], r0_v, sem0)
            idx1 = p1_v[pl.ds(c * CCH, CCH)]
            cp1 = pltpu.async_copy(yr_hbm.at[idx1], r1_v, sem1)
            pltpu.sync_copy(ys_hbm.at[pl.ds(tbase, CCH)], rs_v)
            cp0.wait()
            cp1.wait()
            for i in range(CCH):
                s0 = w0_v[c * CCH + i, :]
                s1 = w1_v[c * CCH + i, :]

                def feat(j, _):
                    sl = pl.ds(j * L, L)
                    out_v[i, sl] = (s0 * r0_v[i, sl] + s1 * r1_v[i, sl]
                                    + rs_v[i, sl])
                    return 0

                lax.fori_loop(0, DIM // L, feat, 0, unroll=4)
            pltpu.sync_copy(out_v, y_hbm.at[pl.ds(tbase, CCH)])

    return k(Yr, Ys, p0f, p1f, w0r, w1r)


@jax.jit
def kernel(x, gate_w, W1, W2, W3, Sw1, Sw2, Sw3):
    b, s, d = x.shape
    xf = x.reshape(-1, d)
    xb16, e0, e1, r0, r1, w0r, w1r, cnt = _gate(xf, gate_w)
    xb16f = xb16
    Ys = _ffn_shared(xb16f, Sw1.astype(jnp.bfloat16),
                     Sw2.astype(jnp.bfloat16), Sw3.astype(jnp.bfloat16))
    p0, p1, tmap, act = _route(cnt, e0, e1, r0, r1)
    p0f = p0.reshape(T)
    p1f = p1.reshape(T)
    tok_src = _sc_scatter(p0f, p1f)
    xi32 = lax.bitcast_convert_type(
        xb16.reshape(T, DIM // 2, 2), jnp.int32)
    Xg32 = _sc_gather(xi32, tok_src)
    Xg = lax.bitcast_convert_type(Xg32, jnp.bfloat16).reshape(R_ROWS, DIM)
    Yr = _ffn_routed(Xg, W1.astype(jnp.bfloat16), W2.astype(jnp.bfloat16),
                     W3.astype(jnp.bfloat16),
                     tmap.reshape(128), act.reshape(128))
    y = _sc_combine(Yr, Ys, p0f, p1f, w0r, w1r)
    return y.reshape(b, s, d)


# speedup vs baseline: 1.5401x; 1.3556x over previous
import functools

import jax
import jax.numpy as jnp
from jax import lax
from jax.experimental import pallas as pl
from jax.experimental.pallas import tpu as pltpu
from jax.experimental.pallas import tpu_sc as plsc

DIM = 1024
HIDDEN = 2752
E = 8
T = 4096

BLK = 512
TILE = 256
R_TILES = T * 2 // TILE + E
R_ROWS = R_TILES * TILE
S_TILES = T // TILE

NC, NS, L = 2, 16, 16
NW = NC * NS


def _gate_body(x_ref, gw_ref, e0_ref, e1_ref, r0_ref, r1_ref,
               w0_ref, w1_ref, cnt_ref, carry):
    i = pl.program_id(0)

    @pl.when(i == 0)
    def _():
        carry[...] = jnp.zeros_like(carry)

    xb = x_ref[...]
    logits = lax.dot_general(xb, gw_ref[...], (((1,), (1,)), ((), ())),
                             preferred_element_type=jnp.float32)
    m = jnp.max(logits, axis=1, keepdims=True)
    ex = jnp.exp(logits - m)
    scores = ex / jnp.sum(ex, axis=1, keepdims=True)
    lane = lax.broadcasted_iota(jnp.int32, scores.shape, 1)
    rank = jnp.zeros(scores.shape, jnp.int32)
    for ep in range(E):
        sc = lax.slice_in_dim(scores, ep, ep + 1, axis=1)
        beats = (sc > scores) | ((sc == scores) & (ep < lane))
        rank = rank + beats.astype(jnp.int32)
    is0 = rank == 0
    is1 = rank == 1
    e0 = jnp.sum(jnp.where(is0, lane, 0), axis=1, keepdims=True)
    e1 = jnp.sum(jnp.where(is1, lane, 0), axis=1, keepdims=True)
    w0 = jnp.sum(jnp.where(is0, scores, 0.0), axis=1, keepdims=True)
    w1 = jnp.sum(jnp.where(is1, scores, 0.0), axis=1, keepdims=True)

    oh = (is0 | is1).astype(jnp.float32)
    tri = (lax.broadcasted_iota(jnp.int32, (BLK, BLK), 0)
           > lax.broadcasted_iota(jnp.int32, (BLK, BLK), 1)).astype(jnp.float32)
    excl = lax.dot_general(tri, oh, (((1,), (0,)), ((), ())),
                           preferred_element_type=jnp.float32)
    cbase = carry[0:1, 0:E]
    r_both = excl + cbase
    r0 = jnp.sum(jnp.where(is0, r_both, 0.0), axis=1, keepdims=True)
    r1 = jnp.sum(jnp.where(is1, r_both, 0.0), axis=1, keepdims=True)

    new_carry = cbase + jnp.sum(oh, axis=0, keepdims=True)
    carry[...] = jnp.pad(new_carry, ((0, 0), (0, 128 - E)))
    cnt_ref[...] = carry[...]

    e0_ref[...] = e0.astype(jnp.int32).reshape(1, BLK // 128, 128)
    e1_ref[...] = e1.astype(jnp.int32).reshape(1, BLK // 128, 128)
    r0_ref[...] = r0.astype(jnp.int32).reshape(1, BLK // 128, 128)
    r1_ref[...] = r1.astype(jnp.int32).reshape(1, BLK // 128, 128)
    w0_ref[...] = jnp.broadcast_to(w0, (BLK, 16))
    w1_ref[...] = jnp.broadcast_to(w1, (BLK, 16))


def _gate(xf, gate_w):
    nblk = T // BLK
    sub = BLK // 128
    return pl.pallas_call(
        _gate_body,
        grid=(nblk,),
        in_specs=[
            pl.BlockSpec((BLK, DIM), lambda i: (i, 0)),
            pl.BlockSpec((E, DIM), lambda i: (0, 0)),
        ],
        out_specs=[
            pl.BlockSpec((1, sub, 128), lambda i: (i, 0, 0)),
            pl.BlockSpec((1, sub, 128), lambda i: (i, 0, 0)),
            pl.BlockSpec((1, sub, 128), lambda i: (i, 0, 0)),
            pl.BlockSpec((1, sub, 128), lambda i: (i, 0, 0)),
            pl.BlockSpec((BLK, 16), lambda i: (i, 0)),
            pl.BlockSpec((BLK, 16), lambda i: (i, 0)),
            pl.BlockSpec((1, 128), lambda i: (0, 0)),
        ],
        out_shape=[
            jax.ShapeDtypeStruct((nblk, sub, 128), jnp.int32),
            jax.ShapeDtypeStruct((nblk, sub, 128), jnp.int32),
            jax.ShapeDtypeStruct((nblk, sub, 128), jnp.int32),
            jax.ShapeDtypeStruct((nblk, sub, 128), jnp.int32),
            jax.ShapeDtypeStruct((T, 16), jnp.float32),
            jax.ShapeDtypeStruct((T, 16), jnp.float32),
            jax.ShapeDtypeStruct((1, 128), jnp.float32),
        ],
        scratch_shapes=[pltpu.VMEM((1, 128), jnp.float32)],
    )(xf, gate_w)


def _route_body(cnt_ref, e0_ref, e1_ref, r0_ref, r1_ref,
                p0_ref, p1_ref, tmap_ref, act_ref):
    counts = cnt_ref[0:1, 0:E].astype(jnp.int32)
    ntiles = (counts + (TILE - 1)) // TILE
    lane8 = lax.broadcasted_iota(jnp.int32, (1, E), 1)
    cumtiles = jnp.zeros((1, E), jnp.int32)
    for ep in range(E):
        nt_e = lax.slice_in_dim(ntiles, ep, ep + 1, axis=1)
        cumtiles = cumtiles + jnp.where(lane8 >= ep, nt_e, 0)
    off_rows = (cumtiles - ntiles) * TILE
    used = lax.slice_in_dim(cumtiles, E - 1, E, axis=1)
    laste = jnp.max(jnp.where(counts > 0, lane8, -1), axis=1, keepdims=True)

    j128 = lax.broadcasted_iota(jnp.int32, (1, 128), 1)
    texp = jnp.zeros((1, 128), jnp.int32)
    for ep in range(E):
        ct_e = lax.slice_in_dim(cumtiles, ep, ep + 1, axis=1)
        texp = texp + (j128 >= ct_e).astype(jnp.int32)
    tmap = jnp.where(j128 >= used, laste, texp)
    tmap_ref[...] = tmap
    act_ref[...] = (j128 < used).astype(jnp.int32)

    e0 = e0_ref[...]
    e1 = e1_ref[...]
    p0 = r0_ref[...]
    p1 = r1_ref[...]
    for ep in range(E):
        off_e = lax.slice_in_dim(off_rows, ep, ep + 1, axis=1)
        p0 = p0 + jnp.where(e0 == ep, off_e, 0)
        p1 = p1 + jnp.where(e1 == ep, off_e, 0)
    p0_ref[...] = p0
    p1_ref[...] = p1


def _route(cnt, e0, e1, r0, r1):
    nblk = T // BLK
    sub = BLK // 128
    return pl.pallas_call(
        _route_body,
        grid=(nblk,),
        in_specs=[
            pl.BlockSpec((1, 128), lambda i: (0, 0)),
            pl.BlockSpec((1, sub, 128), lambda i: (i, 0, 0)),
            pl.BlockSpec((1, sub, 128), lambda i: (i, 0, 0)),
            pl.BlockSpec((1, sub, 128), lambda i: (i, 0, 0)),
            pl.BlockSpec((1, sub, 128), lambda i: (i, 0, 0)),
        ],
        out_specs=[
            pl.BlockSpec((1, sub, 128), lambda i: (i, 0, 0)),
            pl.BlockSpec((1, sub, 128), lambda i: (i, 0, 0)),
            pl.BlockSpec((1, 128), lambda i: (0, 0)),
            pl.BlockSpec((1, 128), lambda i: (0, 0)),
        ],
        out_shape=[
            jax.ShapeDtypeStruct((nblk, sub, 128), jnp.int32),
            jax.ShapeDtypeStruct((nblk, sub, 128), jnp.int32),
            jax.ShapeDtypeStruct((1, 128), jnp.int32),
            jax.ShapeDtypeStruct((1, 128), jnp.int32),
        ],
    )(cnt, e0, e1, r0, r1)


def _sc_scatter(p0f, p1f):
    mesh = plsc.VectorSubcoreMesh(core_axis_name="c", subcore_axis_name="s")

    @functools.partial(
        pl.kernel, mesh=mesh,
        compiler_params=pltpu.CompilerParams(needs_layout_passes=False),
        out_type=jax.ShapeDtypeStruct((R_ROWS,), jnp.int32),
        scratch_types=[
            pltpu.VMEM((R_ROWS,), jnp.int32),
            pltpu.VMEM((T,), jnp.int32),
            pltpu.VMEM((T,), jnp.int32),
        ],
    )
    def k(p0_hbm, p1_hbm, tok_hbm, tok_v, p0_v, p1_v):
        wid = lax.axis_index("s") * NC + lax.axis_index("c")

        @pl.when(wid == 0)
        def _():
            pltpu.sync_copy(p0_hbm, p0_v)
            pltpu.sync_copy(p1_hbm, p1_v)
            iota = lax.iota(jnp.int32, L)
            zeros = jnp.zeros((L,), jnp.int32)

            def init(j, _):
                tok_v[pl.ds(j * L, L)] = zeros
                return 0

            lax.fori_loop(0, R_ROWS // L, init, 0, unroll=False)

            def scat(j, _):
                toks = iota + j * L
                idx0 = p0_v[pl.ds(j * L, L)]
                plsc.store_scatter(tok_v, [idx0], toks)
                idx1 = p1_v[pl.ds(j * L, L)]
                plsc.store_scatter(tok_v, [idx1], toks)
                return 0

            lax.fori_loop(0, T // L, scat, 0, unroll=False)
            pltpu.sync_copy(tok_v, tok_hbm)

    return k(p0f, p1f)


GCH = 40


def _sc_gather(xf, tok_src):
    mesh = plsc.VectorSubcoreMesh(core_axis_name="c", subcore_axis_name="s")
    rows_per_w = R_ROWS // NW
    nch = rows_per_w // GCH

    @functools.partial(
        pl.kernel, mesh=mesh,
        compiler_params=pltpu.CompilerParams(needs_layout_passes=False),
        out_type=jax.ShapeDtypeStruct((R_ROWS, DIM), jnp.float32),
        scratch_types=[
            pltpu.VMEM((GCH,), jnp.int32),
            pltpu.VMEM((GCH,), jnp.int32),
            pltpu.VMEM((GCH, DIM), jnp.float32),
            pltpu.VMEM((GCH, DIM), jnp.float32),
            pltpu.SemaphoreType.DMA,
            pltpu.SemaphoreType.DMA,
            pltpu.SemaphoreType.DMA,
            pltpu.SemaphoreType.DMA,
        ],
    )
    def k(x_hbm, tok_hbm, out_hbm, idx_v0, idx_v1, rows_v0, rows_v1,
          g0, g1, o0, o1):
        wid = lax.axis_index("s") * NC + lax.axis_index("c")
        base = wid * rows_per_w
        idx_v = (idx_v0, idx_v1)
        rows_v = (rows_v0, rows_v1)
        gsem = (g0, g1)
        osem = (o0, o1)
        gathers = [None] * nch
        outs = [None] * nch

        pltpu.sync_copy(tok_hbm.at[pl.ds(base, GCH)], idx_v0)
        gathers[0] = pltpu.async_copy(x_hbm.at[idx_v0], rows_v0, g0)
        for c in range(nch):
            b = c % 2
            nb = (c + 1) % 2
            if c + 1 < nch:
                if c >= 1:
                    outs[c - 1].wait()
                pltpu.sync_copy(tok_hbm.at[pl.ds(base + (c + 1) * GCH, GCH)],
                                idx_v[nb])
                gathers[c + 1] = pltpu.async_copy(x_hbm.at[idx_v[nb]],
                                                  rows_v[nb], gsem[nb])
            gathers[c].wait()
            outs[c] = pltpu.async_copy(
                rows_v[b], out_hbm.at[pl.ds(base + c * GCH, GCH)], osem[b])
        outs[nch - 2].wait()
        outs[nch - 1].wait()

    return k(xf, tok_src)


def _ffn_math(x, w1, w2, w3):
    xb = x.astype(jnp.bfloat16)
    h1 = lax.dot_general(xb, w1, (((1,), (1,)), ((), ())),
                         preferred_element_type=jnp.float32)
    h3 = lax.dot_general(xb, w3, (((1,), (1,)), ((), ())),
                         preferred_element_type=jnp.float32)
    g = (h1 * jax.nn.sigmoid(h1)) * h3
    return lax.dot_general(g.astype(jnp.bfloat16), w2, (((1,), (1,)), ((), ())),
                           preferred_element_type=jnp.float32)


def _ffn_routed_body(tmap_ref, act_ref, x_ref, w1_ref, w2_ref, w3_ref, out_ref):
    t = pl.program_id(0)

    @pl.when(act_ref[t] == 1)
    def _():
        out_ref[...] = _ffn_math(x_ref[...], w1_ref[0], w2_ref[0], w3_ref[0])


def _ffn_routed(Xg, W1b, W2b, W3b, tmap, act):
    grid_spec = pltpu.PrefetchScalarGridSpec(
        num_scalar_prefetch=2,
        grid=(R_TILES,),
        in_specs=[
            pl.BlockSpec((TILE, DIM), lambda t, tm, ac: (t, 0)),
            pl.BlockSpec((1, HIDDEN, DIM), lambda t, tm, ac: (tm[t], 0, 0)),
            pl.BlockSpec((1, DIM, HIDDEN), lambda t, tm, ac: (tm[t], 0, 0)),
            pl.BlockSpec((1, HIDDEN, DIM), lambda t, tm, ac: (tm[t], 0, 0)),
        ],
        out_specs=pl.BlockSpec((TILE, DIM), lambda t, tm, ac: (t, 0)),
    )
    return pl.pallas_call(
        _ffn_routed_body,
        grid_spec=grid_spec,
        out_shape=jax.ShapeDtypeStruct((R_ROWS, DIM), jnp.float32),
    )(tmap, act, Xg, W1b, W2b, W3b)


def _ffn_shared_body(x_ref, w1_ref, w2_ref, w3_ref, out_ref):
    out_ref[...] = _ffn_math(x_ref[...], w1_ref[...], w2_ref[...], w3_ref[...])


def _ffn_shared(xb16f, Sw1b, Sw2b, Sw3b):
    return pl.pallas_call(
        _ffn_shared_body,
        grid=(S_TILES,),
        in_specs=[
            pl.BlockSpec((TILE, DIM), lambda t: (t, 0)),
            pl.BlockSpec((HIDDEN, DIM), lambda t: (0, 0)),
            pl.BlockSpec((DIM, HIDDEN), lambda t: (0, 0)),
            pl.BlockSpec((HIDDEN, DIM), lambda t: (0, 0)),
        ],
        out_specs=pl.BlockSpec((TILE, DIM), lambda t: (t, 0)),
        out_shape=jax.ShapeDtypeStruct((T, DIM), jnp.float32),
    )(xb16f, Sw1b, Sw2b, Sw3b)


CCH = 16


def _sc_combine(Yr, Ys, p0f, p1f, w0r, w1r):
    mesh = plsc.VectorSubcoreMesh(core_axis_name="c", subcore_axis_name="s")
    tok_per_w = T // NW

    @functools.partial(
        pl.kernel, mesh=mesh,
        compiler_params=pltpu.CompilerParams(needs_layout_passes=False),
        out_type=jax.ShapeDtypeStruct((T, DIM), jnp.float32),
        scratch_types=[
            pltpu.VMEM((tok_per_w,), jnp.int32),
            pltpu.VMEM((tok_per_w,), jnp.int32),
            pltpu.VMEM((tok_per_w, 16), jnp.float32),
            pltpu.VMEM((tok_per_w, 16), jnp.float32),
            pltpu.VMEM((CCH, DIM), jnp.float32),
            pltpu.VMEM((CCH, DIM), jnp.float32),
            pltpu.VMEM((CCH, DIM), jnp.float32),
            pltpu.VMEM((CCH, DIM), jnp.float32),
            pltpu.SemaphoreType.DMA,
            pltpu.SemaphoreType.DMA,
        ],
    )
    def k(yr_hbm, ys_hbm, p0_hbm, p1_hbm, w0_hbm, w1_hbm, y_hbm,
          p0_v, p1_v, w0_v, w1_v, r0_v, r1_v, rs_v, out_v, sem0, sem1):
        wid = lax.axis_index("s") * NC + lax.axis_index("c")
        base = wid * tok_per_w
        pltpu.sync_copy(p0_hbm.at[pl.ds(base, tok_per_w)], p0_v)
        pltpu.sync_copy(p1_hbm.at[pl.ds(base, tok_per_w)], p1_v)
        pltpu.sync_copy(w0_hbm.at[pl.ds(base, tok_per_w)], w0_v)
        pltpu.sync_copy(w1_hbm.at[pl.ds(base, tok_per_w)], w1_v)

        for c in range(tok_per_w // CCH):
            tbase = base + c * CCH
            idx0 = p0_v[pl.ds(c * CCH, CCH)]
            cp0 = pltpu.async_copy(yr_hbm.at[idx0], r0_v, sem0)
            idx1 = p1_v[pl.ds(c * CCH, CCH)]
            cp1 = pltpu.async_copy(yr_hbm.at[idx1], r1_v, sem1)
            pltpu.sync_copy(ys_hbm.at[pl.ds(tbase, CCH)], rs_v)
            cp0.wait()
            cp1.wait()
            for i in range(CCH):
                s0 = w0_v[c * CCH + i, :]
                s1 = w1_v[c * CCH + i, :]

                def feat(j, _):
                    sl = pl.ds(j * L, L)
                    out_v[i, sl] = (s0 * r0_v[i, sl] + s1 * r1_v[i, sl]
                                    + rs_v[i, sl])
                    return 0

                lax.fori_loop(0, DIM // L, feat, 0, unroll=4)
            pltpu.sync_copy(out_v, y_hbm.at[pl.ds(tbase, CCH)])

    return k(Yr, Ys, p0f, p1f, w0r, w1r)


@jax.jit
def kernel(x, gate_w, W1, W2, W3, Sw1, Sw2, Sw3):
    b, s, d = x.shape
    xf = x.reshape(-1, d)
    e0, e1, r0, r1, w0r, w1r, cnt = _gate(xf, gate_w)
    Ys = _ffn_shared(xf, Sw1.astype(jnp.bfloat16),
                     Sw2.astype(jnp.bfloat16), Sw3.astype(jnp.bfloat16))
    p0, p1, tmap, act = _route(cnt, e0, e1, r0, r1)
    p0f = p0.reshape(T)
    p1f = p1.reshape(T)
    tok_src = _sc_scatter(p0f, p1f)
    Xg = _sc_gather(xf, tok_src)
    Yr = _ffn_routed(Xg, W1.astype(jnp.bfloat16), W2.astype(jnp.bfloat16),
                     W3.astype(jnp.bfloat16),
                     tmap.reshape(128), act.reshape(128))
    y = _sc_combine(Yr, Ys, p0f, p1f, w0r, w1r)
    return y.reshape(b, s, d)
